# Initial kernel scaffold; baseline (speedup 1.0000x reference)
import jax
import jax.numpy as jnp
from jax.experimental import pallas as pl


def kernel(y, yd, idx_k):
    raise NotImplementedError("write your pallas kernel here")


# speedup vs baseline: 5.3484x; 5.3484x over previous
import functools

import jax
import jax.numpy as jnp
from jax import lax
from jax.experimental import pallas as pl
from jax.experimental.pallas import tpu as pltpu

_EPS = 1e-5


def _fused_body(ydp_ref, yp_ref, idx_ref, out_ref):
    i = pl.program_id(0)

    @pl.when(i == 0)
    def _init():
        out_ref[...] = jnp.zeros_like(out_ref)

    ydp = ydp_ref[...]

    s_sum = jnp.zeros((484, 64), jnp.float32)
    s_sq = jnp.zeros((484, 64), jnp.float32)
    for e in range(36):
        sl = ydp[:, e * 64:(e + 1) * 64]
        s_sum = s_sum + sl
        s_sq = s_sq + sl * sl
    s_mean = s_sum * (1.0 / 36.0)
    s_msq = s_sq * (1.0 / 36.0)

    big = jnp.concatenate([ydp, s_mean, s_msq], axis=1)

    idxv = idx_ref[0]
    iot = lax.broadcasted_iota(jnp.int32, (460, 484), 1)
    P = (idxv == iot).astype(jnp.float32)
    g_all = lax.dot(P, big, precision=lax.Precision.HIGHEST,
                    preferred_element_type=jnp.float32)
    g_feat = g_all[:, :2304]
    s_mean_r = g_all[:, 2304:2368]
    s_msq_r = g_all[:, 2368:2432]

    yp = yp_ref[0]
    c_sum = jnp.zeros((92, 64), jnp.float32)
    c_sq = jnp.zeros((92, 64), jnp.float32)
    for j in range(9):
        sl = yp[:, j * 64:(j + 1) * 64]
        c_sum = c_sum + sl
        c_sq = c_sq + sl * sl
    c_mean = c_sum * (1.0 / 9.0)
    c_std = jnp.sqrt(c_sq * (1.0 / 9.0) - c_mean * c_mean + _EPS)
    c_mean_r = pltpu.repeat(c_mean, 5, 0)
    c_std_r = pltpu.repeat(c_std, 5, 0)

    s_std_r = jnp.sqrt(s_msq_r - s_mean_r * s_mean_r + _EPS)
    a = c_std_r / s_std_r
    b = c_mean_r - s_mean_r * a
    z = g_feat * pltpu.repeat(a, 36, 1) + pltpu.repeat(b, 36, 1)

    for kk in range(5):
        zk = z[kk * 92:(kk + 1) * 92, :].reshape(2, 46, 2304)
        for q in range(4):
            sh, sw = q // 2, q % 2
            for p1 in range(3):
                for p2 in range(3):
                    e = (q * 3 + p1) * 3 + p2
                    gs = zk[:, :, e * 64:(e + 1) * 64]
                    out_ref[sh, pl.ds(2 * i + p1, 2), sw,
                            pl.ds(p2, 46), pl.ds(kk * 64, 64)] += gs

    @pl.when(i == 22)
    def _normalize():
        u = lax.broadcasted_iota(
            jnp.int32, (2, 48, 2, 48, 320), 1).astype(jnp.float32)
        v = lax.broadcasted_iota(
            jnp.int32, (2, 48, 2, 48, 320), 3).astype(jnp.float32)
        cru = jnp.minimum(jnp.minimum(u + 1.0, 48.0 - u), 3.0)
        crv = jnp.minimum(jnp.minimum(v + 1.0, 48.0 - v), 3.0)
        out_ref[...] = out_ref[...] / (cru * crv)


@functools.partial(jax.jit, static_argnames=("interpret",))
def _run(y, yd, idx_k, interpret=False):
    r22 = jnp.arange(22)[:, None] + jnp.arange(3)[None, :]
    t = yd[0].reshape(64, 4, 24, 24)
    t2 = t[:, :, r22[:, :, None, None], r22[None, None, :, :]]
    ydp = t2.transpose(2, 4, 1, 3, 5, 0).reshape(484, 2304)

    r46 = jnp.arange(46)[:, None] + jnp.arange(3)[None, :]
    ty = y[0][:, r46[:, :, None, None], r46[None, None, :, :]]
    yp = ty.transpose(1, 3, 2, 4, 0).reshape(2116, 576).reshape(23, 92, 576)

    idx_r = idx_k[0].reshape(23, 2, 46, 5).transpose(0, 3, 1, 2)
    idx_r = idx_r.reshape(23, 460, 1)

    acc = pl.pallas_call(
        _fused_body,
        grid=(23,),
        in_specs=[
            pl.BlockSpec((484, 2304), lambda i: (0, 0)),
            pl.BlockSpec((1, 92, 576), lambda i: (i, 0, 0)),
            pl.BlockSpec((1, 460, 1), lambda i: (i, 0, 0)),
        ],
        out_specs=pl.BlockSpec((2, 48, 2, 48, 320),
                               lambda i: (0, 0, 0, 0, 0)),
        out_shape=jax.ShapeDtypeStruct((2, 48, 2, 48, 320), jnp.float32),
        interpret=interpret,
    )(ydp, yp, idx_r)

    out = acc.transpose(4, 1, 0, 3, 2).reshape(1, 320, 96, 96)
    return out


def kernel(y, yd, idx_k):
    return _run(y, yd, idx_k)


# speedup vs baseline: 6.8160x; 1.2744x over previous
import functools

import jax
import jax.numpy as jnp
from jax import lax
from jax.experimental import pallas as pl
from jax.experimental.pallas import tpu as pltpu

_EPS = 1e-5


def _fused_body(ydp_ref, yp_ref, idx_ref, out_ref, st_ref):
    i = pl.program_id(0)

    @pl.when(i == 0)
    def _init():
        out_ref[...] = jnp.zeros_like(out_ref)
        ydp0 = ydp_ref[...]
        s_sum = jnp.zeros((484, 64), jnp.float32)
        s_sq = jnp.zeros((484, 64), jnp.float32)
        for e in range(36):
            sl = ydp0[:, e * 64:(e + 1) * 64]
            s_sum = s_sum + sl
            s_sq = s_sq + sl * sl
        st_ref[:, :64] = s_sum * (1.0 / 36.0)
        st_ref[:, 64:] = s_sq * (1.0 / 36.0)

    idxv = idx_ref[0]
    iot = lax.broadcasted_iota(jnp.int32, (460, 484), 1)
    P = (idxv == iot).astype(jnp.float32)
    g_feat = lax.dot(P, ydp_ref[...],
                     preferred_element_type=jnp.float32)
    st = lax.dot(P, st_ref[...], precision=lax.Precision.HIGHEST,
                 preferred_element_type=jnp.float32)
    s_mean_r = st[:, :64]
    s_msq_r = st[:, 64:]

    yp = yp_ref[0]
    c_sum = jnp.zeros((92, 64), jnp.float32)
    c_sq = jnp.zeros((92, 64), jnp.float32)
    for j in range(9):
        sl = yp[:, j * 64:(j + 1) * 64]
        c_sum = c_sum + sl
        c_sq = c_sq + sl * sl
    c_mean = c_sum * (1.0 / 9.0)
    c_std = jnp.sqrt(c_sq * (1.0 / 9.0) - c_mean * c_mean + _EPS)
    c_mean_r = pltpu.repeat(c_mean, 5, 0)
    c_std_r = pltpu.repeat(c_std, 5, 0)

    s_std_r = jnp.sqrt(s_msq_r - s_mean_r * s_mean_r + _EPS)
    a = c_std_r / s_std_r
    b = c_mean_r - s_mean_r * a
    z = g_feat * pltpu.repeat(a, 36, 1) + pltpu.repeat(b, 36, 1)

    for kk in range(5):
        zk = z[kk * 92:(kk + 1) * 92, :].reshape(2, 46, 2304)
        for q in range(4):
            sh, sw = q // 2, q % 2
            for p1 in range(3):
                for p2 in range(3):
                    e = (q * 3 + p1) * 3 + p2
                    gs = zk[:, :, e * 64:(e + 1) * 64]
                    out_ref[sh, pl.ds(2 * i + p1, 2), sw,
                            pl.ds(p2, 46), pl.ds(kk * 64, 64)] += gs

    @pl.when(i == 22)
    def _normalize():
        u = lax.broadcasted_iota(
            jnp.int32, (2, 48, 2, 48, 320), 1).astype(jnp.float32)
        v = lax.broadcasted_iota(
            jnp.int32, (2, 48, 2, 48, 320), 3).astype(jnp.float32)
        cru = jnp.minimum(jnp.minimum(u + 1.0, 48.0 - u), 3.0)
        crv = jnp.minimum(jnp.minimum(v + 1.0, 48.0 - v), 3.0)
        out_ref[...] = out_ref[...] / (cru * crv)


@functools.partial(jax.jit, static_argnames=("interpret",))
def _run(y, yd, idx_k, interpret=False):
    r22 = jnp.arange(22)[:, None] + jnp.arange(3)[None, :]
    t = yd[0].reshape(64, 4, 24, 24)
    t2 = t[:, :, r22[:, :, None, None], r22[None, None, :, :]]
    ydp = t2.transpose(2, 4, 1, 3, 5, 0).reshape(484, 2304)

    r46 = jnp.arange(46)[:, None] + jnp.arange(3)[None, :]
    ty = y[0][:, r46[:, :, None, None], r46[None, None, :, :]]
    yp = ty.transpose(1, 3, 2, 4, 0).reshape(2116, 576).reshape(23, 92, 576)

    idx_r = idx_k[0].reshape(23, 2, 46, 5).transpose(0, 3, 1, 2)
    idx_r = idx_r.reshape(23, 460, 1)

    acc = pl.pallas_call(
        _fused_body,
        grid=(23,),
        in_specs=[
            pl.BlockSpec((484, 2304), lambda i: (0, 0)),
            pl.BlockSpec((1, 92, 576), lambda i: (i, 0, 0)),
            pl.BlockSpec((1, 460, 1), lambda i: (i, 0, 0)),
        ],
        out_specs=pl.BlockSpec((2, 48, 2, 48, 320),
                               lambda i: (0, 0, 0, 0, 0)),
        out_shape=jax.ShapeDtypeStruct((2, 48, 2, 48, 320), jnp.float32),
        scratch_shapes=[pltpu.VMEM((484, 128), jnp.float32)],
        interpret=interpret,
    )(ydp, yp, idx_r)

    out = acc.transpose(4, 1, 0, 3, 2).reshape(1, 320, 96, 96)
    return out


def kernel(y, yd, idx_k):
    return _run(y, yd, idx_k)


# speedup vs baseline: 6.8164x; 1.0001x over previous
import functools

import jax
import jax.numpy as jnp
from jax import lax
from jax.experimental import pallas as pl
from jax.experimental.pallas import tpu as pltpu

_EPS = 1e-5


def _fused_body(ydp_ref, yt_ref, idx_ref, out_ref, st_ref):
    i = pl.program_id(0)

    @pl.when(i == 0)
    def _init():
        out_ref[...] = jnp.zeros_like(out_ref)
        ydp0 = ydp_ref[...]
        s_sum = jnp.zeros((484, 64), jnp.float32)
        s_sq = jnp.zeros((484, 64), jnp.float32)
        for e in range(36):
            sl = ydp0[:, e * 64:(e + 1) * 64]
            s_sum = s_sum + sl
            s_sq = s_sq + sl * sl
        st_ref[:, :64] = s_sum * (1.0 / 36.0)
        st_ref[:, 64:] = s_sq * (1.0 / 36.0)

    idxv = idx_ref[0]
    iot = lax.broadcasted_iota(jnp.int32, (460, 484), 1)
    P = (idxv == iot).astype(jnp.float32)
    g_feat = lax.dot(P, ydp_ref[...],
                     preferred_element_type=jnp.float32)
    st = lax.dot(P, st_ref[...], precision=lax.Precision.HIGHEST,
                 preferred_element_type=jnp.float32)
    s_mean_r = st[:, :64]
    s_msq_r = st[:, 64:]

    yp = yt_ref[0]
    c_sum = jnp.zeros((92, 64), jnp.float32)
    c_sq = jnp.zeros((92, 64), jnp.float32)
    for j in range(9):
        sl = yp[:, j * 64:(j + 1) * 64]
        c_sum = c_sum + sl
        c_sq = c_sq + sl * sl
    c_mean = c_sum * (1.0 / 9.0)
    c_std = jnp.sqrt(c_sq * (1.0 / 9.0) - c_mean * c_mean + _EPS)
    c_mean_r = pltpu.repeat(c_mean, 5, 0)
    c_std_r = pltpu.repeat(c_std, 5, 0)

    s_std_r = jnp.sqrt(s_msq_r - s_mean_r * s_mean_r + _EPS)
    a = c_std_r / s_std_r
    b = c_mean_r - s_mean_r * a
    z = g_feat * pltpu.repeat(a, 36, 1) + pltpu.repeat(b, 36, 1)

    for kk in range(5):
        zk = z[kk * 92:(kk + 1) * 92, :].reshape(2, 46, 2304)
        for q in range(4):
            sh, sw = q // 2, q % 2
            for p1 in range(3):
                for p2 in range(3):
                    e = (q * 3 + p1) * 3 + p2
                    gs = zk[:, :, e * 64:(e + 1) * 64]
                    out_ref[sh, pl.ds(2 * i + p1, 2), sw,
                            pl.ds(p2, 46), pl.ds(kk * 64, 64)] += gs

    @pl.when(i == 22)
    def _normalize():
        u = lax.broadcasted_iota(
            jnp.int32, (2, 48, 2, 48, 320), 1).astype(jnp.float32)
        v = lax.broadcasted_iota(
            jnp.int32, (2, 48, 2, 48, 320), 3).astype(jnp.float32)
        cru = jnp.minimum(jnp.minimum(u + 1.0, 48.0 - u), 3.0)
        crv = jnp.minimum(jnp.minimum(v + 1.0, 48.0 - v), 3.0)
        out_ref[...] = out_ref[...] / (cru * crv)


@functools.partial(jax.jit, static_argnames=("interpret",))
def _run(y, yd, idx_k, interpret=False):
    r22 = jnp.arange(22)[:, None] + jnp.arange(3)[None, :]
    t = yd[0].reshape(64, 4, 24, 24)
    t2 = t[:, :, r22[:, :, None, None], r22[None, None, :, :]]
    ydp = t2.transpose(2, 4, 1, 3, 5, 0).reshape(484, 2304)

    r46 = jnp.arange(46)[:, None] + jnp.arange(3)[None, :]
    ty = y[0][:, r46[:, :, None, None], r46[None, None, :, :]]
    yt = ty.transpose(1, 3, 2, 4, 0).reshape(23, 92, 576)

    idx_r = idx_k[0].reshape(23, 2, 46, 5).transpose(0, 3, 1, 2)
    idx_r = idx_r.reshape(23, 460, 1)

    acc = pl.pallas_call(
        _fused_body,
        grid=(23,),
        in_specs=[
            pl.BlockSpec((484, 2304), lambda i: (0, 0)),
            pl.BlockSpec((1, 92, 576), lambda i: (i, 0, 0)),
            pl.BlockSpec((1, 460, 1), lambda i: (i, 0, 0)),
        ],
        out_specs=pl.BlockSpec((2, 48, 2, 48, 320),
                               lambda i: (0, 0, 0, 0, 0)),
        out_shape=jax.ShapeDtypeStruct((2, 48, 2, 48, 320), jnp.float32),
        scratch_shapes=[pltpu.VMEM((484, 128), jnp.float32)],
        interpret=interpret,
    )(ydp, yt, idx_r)

    out = acc.transpose(4, 1, 0, 3, 2).reshape(1, 320, 96, 96)
    return out


def kernel(y, yd, idx_k):
    return _run(y, yd, idx_k)


# speedup vs baseline: 11.4981x; 1.6868x over previous
import functools

import jax
import jax.numpy as jnp
from jax import lax
from jax.experimental import pallas as pl
from jax.experimental.pallas import tpu as pltpu

_EPS = 1e-5


def _transpose_body(acc_ref, out_ref):
    m = acc_ref[0, :, 0, :, :].reshape(2304, 320)
    out_ref[:, 0, 0, :, :] = m.T.reshape(320, 48, 48)


def _fused_body(ydp_ref, yt_ref, idx_ref, acc_ref, st_ref):
    i = pl.program_id(0)

    @pl.when(i == 0)
    def _init():
        acc_ref[...] = jnp.zeros_like(acc_ref)
        ydp0 = ydp_ref[...]
        s_sum = jnp.zeros((484, 64), jnp.float32)
        s_sq = jnp.zeros((484, 64), jnp.float32)
        for e in range(36):
            sl = ydp0[:, e * 64:(e + 1) * 64]
            s_sum = s_sum + sl
            s_sq = s_sq + sl * sl
        st_ref[:, :64] = s_sum * (1.0 / 36.0)
        st_ref[:, 64:] = s_sq * (1.0 / 36.0)

    idxv = idx_ref[0]
    iot = lax.broadcasted_iota(jnp.int32, (460, 484), 1)
    P = (idxv == iot).astype(jnp.float32)
    g_feat = lax.dot(P, ydp_ref[...],
                     preferred_element_type=jnp.float32)
    st = lax.dot(P, st_ref[...], precision=lax.Precision.HIGHEST,
                 preferred_element_type=jnp.float32)
    s_mean_r = st[:, :64]
    s_msq_r = st[:, 64:]

    yp = yt_ref[0]
    c_sum = jnp.zeros((92, 64), jnp.float32)
    c_sq = jnp.zeros((92, 64), jnp.float32)
    for j in range(9):
        sl = yp[:, j * 64:(j + 1) * 64]
        c_sum = c_sum + sl
        c_sq = c_sq + sl * sl
    c_mean = c_sum * (1.0 / 9.0)
    c_std = jnp.sqrt(c_sq * (1.0 / 9.0) - c_mean * c_mean + _EPS)
    c_mean_r = pltpu.repeat(c_mean, 5, 0)
    c_std_r = pltpu.repeat(c_std, 5, 0)

    s_std_r = jnp.sqrt(s_msq_r - s_mean_r * s_mean_r + _EPS)
    a = c_std_r / s_std_r
    b = c_mean_r - s_mean_r * a
    z = g_feat * pltpu.repeat(a, 36, 1) + pltpu.repeat(b, 36, 1)

    for kk in range(5):
        zk = z[kk * 92:(kk + 1) * 92, :].reshape(2, 46, 2304)
        for q in range(4):
            sh, sw = q // 2, q % 2
            for p1 in range(3):
                for p2 in range(3):
                    e = (q * 3 + p1) * 3 + p2
                    gs = zk[:, :, e * 64:(e + 1) * 64]
                    acc_ref[sh, pl.ds(2 * i + p1, 2), sw,
                            pl.ds(p2, 46), pl.ds(kk * 64, 64)] += gs

    @pl.when(i == 22)
    def _normalize():
        u = lax.broadcasted_iota(
            jnp.int32, (2, 48, 2, 48, 320), 1).astype(jnp.float32)
        v = lax.broadcasted_iota(
            jnp.int32, (2, 48, 2, 48, 320), 3).astype(jnp.float32)
        cru = jnp.minimum(jnp.minimum(u + 1.0, 48.0 - u), 3.0)
        crv = jnp.minimum(jnp.minimum(v + 1.0, 48.0 - v), 3.0)
        acc_ref[...] = acc_ref[...] / (cru * crv)


@functools.partial(jax.jit, static_argnames=("interpret",))
def _run(y, yd, idx_k, interpret=False):
    r22 = jnp.arange(22)[:, None] + jnp.arange(3)[None, :]
    t = yd[0].reshape(64, 4, 24, 24)
    t2 = t[:, :, r22[:, :, None, None], r22[None, None, :, :]]
    ydp = t2.transpose(2, 4, 1, 3, 5, 0).reshape(484, 2304)

    r46 = jnp.arange(46)[:, None] + jnp.arange(3)[None, :]
    ty = y[0][:, r46[:, :, None, None], r46[None, None, :, :]]
    yt = ty.transpose(1, 3, 2, 4, 0).reshape(23, 92, 576)

    idx_r = idx_k[0].reshape(23, 2, 46, 5).transpose(0, 3, 1, 2)
    idx_r = idx_r.reshape(23, 460, 1)

    acc = pl.pallas_call(
        _fused_body,
        grid=(23,),
        in_specs=[
            pl.BlockSpec((484, 2304), lambda i: (0, 0)),
            pl.BlockSpec((1, 92, 576), lambda i: (i, 0, 0)),
            pl.BlockSpec((1, 460, 1), lambda i: (i, 0, 0)),
        ],
        out_specs=pl.BlockSpec((2, 48, 2, 48, 320),
                               lambda i: (0, 0, 0, 0, 0)),
        out_shape=jax.ShapeDtypeStruct((2, 48, 2, 48, 320), jnp.float32),
        scratch_shapes=[pltpu.VMEM((484, 128), jnp.float32)],
        interpret=interpret,
    )(ydp, yt, idx_r)

    out2 = pl.pallas_call(
        _transpose_body,
        grid=(2, 2),
        in_specs=[pl.BlockSpec((1, 48, 1, 48, 320),
                               lambda sh, sw: (sh, 0, sw, 0, 0))],
        out_specs=pl.BlockSpec((320, 1, 1, 48, 48),
                               lambda sh, sw: (0, sh, sw, 0, 0)),
        out_shape=jax.ShapeDtypeStruct((320, 2, 2, 48, 48), jnp.float32),
        interpret=interpret,
    )(acc)

    out = out2.transpose(0, 3, 1, 4, 2).reshape(1, 320, 96, 96)
    return out


def kernel(y, yd, idx_k):
    return _run(y, yd, idx_k)


# speedup vs baseline: 11.5039x; 1.0005x over previous
import functools

import jax
import jax.numpy as jnp
from jax import lax
from jax.experimental import pallas as pl
from jax.experimental.pallas import tpu as pltpu

_EPS = 1e-5


def _transpose_body(acc_ref, out_ref):
    m = acc_ref[0, :, 0, :, :].reshape(2304, 320)
    out_ref[:, 0, 0, :, :] = m.T.reshape(320, 48, 48)


def _fused_body(yd4_ref, y_ref, idx_ref, acc_ref, ydp_ref, st_ref):
    i = pl.program_id(0)

    @pl.when(i == 0)
    def _init():
        acc_ref[...] = jnp.zeros_like(acc_ref)
        for q in range(4):
            for p1 in range(3):
                for p2 in range(3):
                    e = (q * 3 + p1) * 3 + p2
                    for r in range(22):
                        ydp_ref[r * 22:(r + 1) * 22,
                                e * 64:(e + 1) * 64] = (
                            yd4_ref[q, r + p1, p2:p2 + 22, :])
        ydp0 = ydp_ref[...]
        s_sum = jnp.zeros((484, 64), jnp.float32)
        s_sq = jnp.zeros((484, 64), jnp.float32)
        for e in range(36):
            sl = ydp0[:, e * 64:(e + 1) * 64]
            s_sum = s_sum + sl
            s_sq = s_sq + sl * sl
        st_ref[:, :64] = s_sum * (1.0 / 36.0)
        st_ref[:, 64:] = s_sq * (1.0 / 36.0)

    idxv = idx_ref[0]
    iot = lax.broadcasted_iota(jnp.int32, (460, 484), 1)
    P = (idxv == iot).astype(jnp.float32)
    g_feat = lax.dot(P, ydp_ref[...],
                     preferred_element_type=jnp.float32)
    st = lax.dot(P, st_ref[...], precision=lax.Precision.HIGHEST,
                 preferred_element_type=jnp.float32)
    s_mean_r = st[:, :64]
    s_msq_r = st[:, 64:]

    ysl = y_ref[:, pl.ds(2 * i, 4), :]
    cs = []
    for mr in range(2):
        c_sum = jnp.zeros((64, 46), jnp.float32)
        c_sq = jnp.zeros((64, 46), jnp.float32)
        for p1 in range(3):
            for p2 in range(3):
                sl = ysl[:, mr + p1, p2:p2 + 46]
                c_sum = c_sum + sl
                c_sq = c_sq + sl * sl
        cs.append((c_sum, c_sq))
    c_sum_t = jnp.concatenate([cs[0][0], cs[1][0]], axis=1)
    c_sq_t = jnp.concatenate([cs[0][1], cs[1][1]], axis=1)
    c_mean_t = c_sum_t * (1.0 / 9.0)
    c_std_t = jnp.sqrt(c_sq_t * (1.0 / 9.0) - c_mean_t * c_mean_t + _EPS)
    c_mean = c_mean_t.T
    c_std = c_std_t.T
    c_mean_r = pltpu.repeat(c_mean, 5, 0)
    c_std_r = pltpu.repeat(c_std, 5, 0)

    s_std_r = jnp.sqrt(s_msq_r - s_mean_r * s_mean_r + _EPS)
    a = c_std_r / s_std_r
    b = c_mean_r - s_mean_r * a
    z = g_feat * pltpu.repeat(a, 36, 1) + pltpu.repeat(b, 36, 1)

    for kk in range(5):
        zk = z[kk * 92:(kk + 1) * 92, :].reshape(2, 46, 2304)
        for q in range(4):
            sh, sw = q // 2, q % 2
            for p1 in range(3):
                for p2 in range(3):
                    e = (q * 3 + p1) * 3 + p2
                    gs = zk[:, :, e * 64:(e + 1) * 64]
                    acc_ref[sh, pl.ds(2 * i + p1, 2), sw,
                            pl.ds(p2, 46), pl.ds(kk * 64, 64)] += gs

    @pl.when(i == 22)
    def _normalize():
        u = lax.broadcasted_iota(
            jnp.int32, (2, 48, 2, 48, 320), 1).astype(jnp.float32)
        v = lax.broadcasted_iota(
            jnp.int32, (2, 48, 2, 48, 320), 3).astype(jnp.float32)
        cru = jnp.minimum(jnp.minimum(u + 1.0, 48.0 - u), 3.0)
        crv = jnp.minimum(jnp.minimum(v + 1.0, 48.0 - v), 3.0)
        acc_ref[...] = acc_ref[...] / (cru * crv)


@functools.partial(jax.jit, static_argnames=("interpret",))
def _run(y, yd, idx_k, interpret=False):
    yd4 = yd[0].reshape(64, 4, 24, 24).transpose(1, 2, 3, 0)
    yr = y[0]
    idx_r = idx_k[0].reshape(23, 2, 46, 5).transpose(0, 3, 1, 2)
    idx_r = idx_r.reshape(23, 460, 1)

    acc = pl.pallas_call(
        _fused_body,
        grid=(23,),
        in_specs=[
            pl.BlockSpec((4, 24, 24, 64), lambda i: (0, 0, 0, 0)),
            pl.BlockSpec((64, 48, 48), lambda i: (0, 0, 0)),
            pl.BlockSpec((1, 460, 1), lambda i: (i, 0, 0)),
        ],
        out_specs=pl.BlockSpec((2, 48, 2, 48, 320),
                               lambda i: (0, 0, 0, 0, 0)),
        out_shape=jax.ShapeDtypeStruct((2, 48, 2, 48, 320), jnp.float32),
        scratch_shapes=[pltpu.VMEM((484, 2304), jnp.float32),
                        pltpu.VMEM((484, 128), jnp.float32)],
        interpret=interpret,
    )(yd4, yr, idx_r)

    out2 = pl.pallas_call(
        _transpose_body,
        grid=(2, 2),
        in_specs=[pl.BlockSpec((1, 48, 1, 48, 320),
                               lambda sh, sw: (sh, 0, sw, 0, 0))],
        out_specs=pl.BlockSpec((320, 1, 1, 48, 48),
                               lambda sh, sw: (0, sh, sw, 0, 0)),
        out_shape=jax.ShapeDtypeStruct((320, 2, 2, 48, 48), jnp.float32),
        interpret=interpret,
    )(acc)

    out = out2.transpose(0, 3, 1, 4, 2).reshape(1, 320, 96, 96)
    return out


def kernel(y, yd, idx_k):
    return _run(y, yd, idx_k)


# speedup vs baseline: 14.9579x; 1.3002x over previous
import functools

import jax
import jax.numpy as jnp
from jax import lax
from jax.experimental import pallas as pl
from jax.experimental.pallas import tpu as pltpu

_EPS = 1e-5


def _transpose_body(acc_ref, out_ref):
    row = lax.broadcasted_iota(jnp.int32, (96, 96), 0)
    col = lax.broadcasted_iota(jnp.int32, (96, 96), 1)
    tgt = jnp.where(row < 48, 2 * row, 2 * (row - 48) + 1)
    perm = (col == tgt).astype(jnp.float32)
    ys = []
    for sh in range(2):
        xs = []
        for sw in range(2):
            ts = [acc_ref[sh, u, sw, :, :].reshape(48, 320)
                  .T.reshape(320, 1, 48) for u in range(4)]
            xs.append(jnp.concatenate(ts, axis=1).reshape(1280, 48))
        x = jnp.concatenate(xs, axis=1)
        y = lax.dot(x, perm, precision=lax.Precision.HIGHEST,
                    preferred_element_type=jnp.float32)
        ys.append(y.reshape(320, 4, 1, 96))
    out_ref[...] = jnp.concatenate(ys, axis=2).reshape(320, 8, 96)


def _fused_body(yd4_ref, y_ref, idx_ref, acc_ref, ydp_ref, st_ref):
    i = pl.program_id(0)

    @pl.when(i == 0)
    def _init():
        acc_ref[...] = jnp.zeros_like(acc_ref)
        for q in range(4):
            for p1 in range(3):
                for p2 in range(3):
                    e = (q * 3 + p1) * 3 + p2
                    for r in range(22):
                        ydp_ref[r * 22:(r + 1) * 22,
                                e * 64:(e + 1) * 64] = (
                            yd4_ref[q, r + p1, p2:p2 + 22, :])
        ydp0 = ydp_ref[...]
        s_sum = jnp.zeros((484, 64), jnp.float32)
        s_sq = jnp.zeros((484, 64), jnp.float32)
        for e in range(36):
            sl = ydp0[:, e * 64:(e + 1) * 64]
            s_sum = s_sum + sl
            s_sq = s_sq + sl * sl
        st_ref[:, :64] = s_sum * (1.0 / 36.0)
        st_ref[:, 64:] = s_sq * (1.0 / 36.0)

    idxv = idx_ref[0]
    iot = lax.broadcasted_iota(jnp.int32, (460, 484), 1)
    P = (idxv == iot).astype(jnp.float32)
    g_feat = lax.dot(P, ydp_ref[...],
                     preferred_element_type=jnp.float32)
    st = lax.dot(P, st_ref[...], precision=lax.Precision.HIGHEST,
                 preferred_element_type=jnp.float32)
    s_mean_r = st[:, :64]
    s_msq_r = st[:, 64:]

    ysl = y_ref[:, pl.ds(2 * i, 4), :]
    cs = []
    for mr in range(2):
        c_sum = jnp.zeros((64, 46), jnp.float32)
        c_sq = jnp.zeros((64, 46), jnp.float32)
        for p1 in range(3):
            for p2 in range(3):
                sl = ysl[:, mr + p1, p2:p2 + 46]
                c_sum = c_sum + sl
                c_sq = c_sq + sl * sl
        cs.append((c_sum, c_sq))
    c_sum_t = jnp.concatenate([cs[0][0], cs[1][0]], axis=1)
    c_sq_t = jnp.concatenate([cs[0][1], cs[1][1]], axis=1)
    c_mean_t = c_sum_t * (1.0 / 9.0)
    c_std_t = jnp.sqrt(c_sq_t * (1.0 / 9.0) - c_mean_t * c_mean_t + _EPS)
    c_mean = c_mean_t.T
    c_std = c_std_t.T
    c_mean_r = pltpu.repeat(c_mean, 5, 0)
    c_std_r = pltpu.repeat(c_std, 5, 0)

    s_std_r = jnp.sqrt(s_msq_r - s_mean_r * s_mean_r + _EPS)
    a = c_std_r / s_std_r
    b = c_mean_r - s_mean_r * a
    z = g_feat * pltpu.repeat(a, 36, 1) + pltpu.repeat(b, 36, 1)

    for kk in range(5):
        zk = z[kk * 92:(kk + 1) * 92, :].reshape(2, 46, 2304)
        for q in range(4):
            sh, sw = q // 2, q % 2
            for p1 in range(3):
                for p2 in range(3):
                    e = (q * 3 + p1) * 3 + p2
                    gs = zk[:, :, e * 64:(e + 1) * 64]
                    acc_ref[sh, pl.ds(2 * i + p1, 2), sw,
                            pl.ds(p2, 46), pl.ds(kk * 64, 64)] += gs

    @pl.when(i == 22)
    def _normalize():
        u = lax.broadcasted_iota(
            jnp.int32, (2, 48, 2, 48, 320), 1).astype(jnp.float32)
        v = lax.broadcasted_iota(
            jnp.int32, (2, 48, 2, 48, 320), 3).astype(jnp.float32)
        cru = jnp.minimum(jnp.minimum(u + 1.0, 48.0 - u), 3.0)
        crv = jnp.minimum(jnp.minimum(v + 1.0, 48.0 - v), 3.0)
        acc_ref[...] = acc_ref[...] / (cru * crv)


@functools.partial(jax.jit, static_argnames=("interpret",))
def _run(y, yd, idx_k, interpret=False):
    yd4 = yd[0].reshape(64, 4, 24, 24).transpose(1, 2, 3, 0)
    yr = y[0]
    idx_r = idx_k[0].reshape(23, 2, 46, 5).transpose(0, 3, 1, 2)
    idx_r = idx_r.reshape(23, 460, 1)

    acc = pl.pallas_call(
        _fused_body,
        grid=(23,),
        in_specs=[
            pl.BlockSpec((4, 24, 24, 64), lambda i: (0, 0, 0, 0)),
            pl.BlockSpec((64, 48, 48), lambda i: (0, 0, 0)),
            pl.BlockSpec((1, 460, 1), lambda i: (i, 0, 0)),
        ],
        out_specs=pl.BlockSpec((2, 48, 2, 48, 320),
                               lambda i: (0, 0, 0, 0, 0)),
        out_shape=jax.ShapeDtypeStruct((2, 48, 2, 48, 320), jnp.float32),
        scratch_shapes=[pltpu.VMEM((484, 2304), jnp.float32),
                        pltpu.VMEM((484, 128), jnp.float32)],
        interpret=interpret,
    )(yd4, yr, idx_r)

    out2 = pl.pallas_call(
        _transpose_body,
        grid=(12,),
        in_specs=[pl.BlockSpec((2, 4, 2, 48, 320),
                               lambda t: (0, t, 0, 0, 0))],
        out_specs=pl.BlockSpec((320, 8, 96), lambda t: (0, t, 0)),
        out_shape=jax.ShapeDtypeStruct((320, 96, 96), jnp.float32),
        interpret=interpret,
    )(acc)

    return out2[None]


def kernel(y, yd, idx_k):
    return _run(y, yd, idx_k)
